# baseline (speedup 1.0000x reference)
import functools

import jax
import jax.numpy as jnp
from jax import lax
from jax.experimental import pallas as pl
from jax.experimental.pallas import tpu as pltpu
from jax.experimental.pallas import tpu_sc as plsc

B = 16384
N_CONT = 13
N_CAT = 26
V1 = 1001
D = 128

NC = 2
NS = 16
NW = NC * NS
BPW = B // NW
S = 2
ROWS = S * N_CAT
CH = BPW // S
NBUF = 4
HALF = BPW // 2
CHH = CH // 2


def _sc_body(tab_hbm, fidx_hbm, out_hbm, idx_v, ring_v, out_v, sem0, sem1, sem2, sem3):
    wid = lax.axis_index("s") * NC + lax.axis_index("c")
    pltpu.sync_copy(fidx_hbm.at[wid], idx_v)
    sems = (sem0, sem1, sem2, sem3)

    def start(c, buf):
        pltpu.async_copy(tab_hbm.at[idx_v.at[c]], ring_v.at[buf], sems[buf])

    def wait(buf):
        pltpu.make_async_copy(
            tab_hbm.at[idx_v.at[0]], ring_v.at[buf], sems[buf]
        ).wait()

    def accum(c, buf):
        for s in range(S):
            acc = [ring_v[buf, s * N_CAT, pl.ds(v * 16, 16)] for v in range(8)]
            for i in range(1, N_CAT):
                for v in range(8):
                    acc[v] = acc[v] + ring_v[buf, s * N_CAT + i, pl.ds(v * 16, 16)]
            row = pl.multiple_of((c * S + s) * D, 128)
            for v in range(8):
                out_v[pl.ds(row + v * 16, 16)] = acc[v]

    for h in range(2):
        hc = h * CHH
        for b in range(NBUF):
            start(hc + b, b)

        def rot(p, carry, hc=hc):
            c0 = NBUF * p
            for b in range(NBUF):
                wait(b)
                accum(c0 + b, b)

                @pl.when(c0 + b + NBUF < CHH)
                def _():
                    start(hc + c0 + b + NBUF, b)

            return carry

        lax.fori_loop(0, CHH // NBUF, rot, None)
        pltpu.sync_copy(out_v, out_hbm.at[2 * wid + h])


_sc_embed = functools.partial(
    pl.kernel,
    out_type=jax.ShapeDtypeStruct((2 * NW, HALF * D), jnp.float32),
    mesh=plsc.VectorSubcoreMesh(core_axis_name="c", subcore_axis_name="s"),
    scratch_types=[
        pltpu.VMEM((CH, ROWS), jnp.int32),
        pltpu.VMEM((NBUF, ROWS, D), jnp.float32),
        pltpu.VMEM((HALF * D,), jnp.float32),
        pltpu.SemaphoreType.DMA,
        pltpu.SemaphoreType.DMA,
        pltpu.SemaphoreType.DMA,
        pltpu.SemaphoreType.DMA,
    ],
)(_sc_body)


BLK = 2048


def _dense_body(cont_ref, cat_ref, wc_ref, bc_ref, wo_ref, bo_ref, out_ref):
    c1 = jnp.dot(cont_ref[...], wc_ref[...], preferred_element_type=jnp.float32)
    c1 = c1 + bc_ref[...]
    wo = wo_ref[...]
    h = jnp.dot(c1, wo[:D], preferred_element_type=jnp.float32)
    h = h + jnp.dot(cat_ref[...], wo[D:], preferred_element_type=jnp.float32)
    out_ref[...] = h + bo_ref[...]


def _dense(cont, cat_sum, W_cont, b_cont, W_out, b_out):
    return pl.pallas_call(
        _dense_body,
        grid=(B // BLK,),
        in_specs=[
            pl.BlockSpec((BLK, N_CONT), lambda i: (i, 0)),
            pl.BlockSpec((BLK, D), lambda i: (i, 0)),
            pl.BlockSpec((N_CONT, D), lambda i: (0, 0)),
            pl.BlockSpec((1, D), lambda i: (0, 0)),
            pl.BlockSpec((2 * D, D), lambda i: (0, 0)),
            pl.BlockSpec((1, D), lambda i: (0, 0)),
        ],
        out_specs=pl.BlockSpec((BLK, D), lambda i: (i, 0)),
        out_shape=jax.ShapeDtypeStruct((B, D), jnp.float32),
    )(cont, cat_sum, W_cont, b_cont, W_out, b_out)


def kernel(x, W_cont, b_cont, tables, W_out, b_out):
    x = x.astype(jnp.int32)
    cont = x[:, :N_CONT].astype(jnp.float32)
    offs = 1 + V1 * jnp.arange(N_CAT, dtype=jnp.int32)
    fidx = (x[:, N_CONT:] + offs).reshape(NW, CH, ROWS)
    tab_flat = tables.reshape(N_CAT * V1, D)
    cat_sum = _sc_embed(tab_flat, fidx).reshape(B, D)
    return _dense(
        cont, cat_sum, W_cont, b_cont.reshape(1, D), W_out, b_out.reshape(1, D)
    )


# speedup vs baseline: 1.1231x; 1.1231x over previous
import functools

import jax
import jax.numpy as jnp
from jax import lax
from jax.experimental import pallas as pl
from jax.experimental.pallas import tpu as pltpu
from jax.experimental.pallas import tpu_sc as plsc

B = 16384
N_CONT = 13
N_CAT = 26
V1 = 1001
D = 128

NSC = 16
NTC = N_CAT - NSC
VP = 1024

NC = 2
NS = 16
NW = NC * NS
BPW = B // NW
S = 4
ROWS = S * NSC
CH = BPW // S
NBUF = 4
HALF = BPW // 2
CHH = CH // 2


def _sc_body(tab_hbm, fidx_hbm, out_hbm, idx_v, ring_v, out_v, sem0, sem1, sem2, sem3):
    wid = lax.axis_index("s") * NC + lax.axis_index("c")
    pltpu.sync_copy(fidx_hbm.at[wid], idx_v)
    sems = (sem0, sem1, sem2, sem3)

    def start(c, buf):
        pltpu.async_copy(tab_hbm.at[idx_v.at[c]], ring_v.at[buf], sems[buf])

    def wait(buf):
        pltpu.make_async_copy(
            tab_hbm.at[idx_v.at[0]], ring_v.at[buf], sems[buf]
        ).wait()

    def accum(c, buf):
        for s in range(S):
            acc = [ring_v[buf, s * NSC, pl.ds(v * 16, 16)] for v in range(8)]
            for i in range(1, NSC):
                for v in range(8):
                    acc[v] = acc[v] + ring_v[buf, s * NSC + i, pl.ds(v * 16, 16)]
            row = pl.multiple_of((c * S + s) * D, 128)
            for v in range(8):
                out_v[pl.ds(row + v * 16, 16)] = acc[v]

    for h in range(2):
        hc = h * CHH
        for b in range(NBUF):
            start(hc + b, b)

        def rot(p, carry, hc=hc):
            c0 = NBUF * p
            for b in range(NBUF):
                wait(b)
                accum(c0 + b, b)

                @pl.when(c0 + b + NBUF < CHH)
                def _():
                    start(hc + c0 + b + NBUF, b)

            return carry

        lax.fori_loop(0, CHH // NBUF, rot, None)
        pltpu.sync_copy(out_v, out_hbm.at[2 * wid + h])


_sc_embed = functools.partial(
    pl.kernel,
    out_type=jax.ShapeDtypeStruct((2 * NW, HALF * D), jnp.float32),
    mesh=plsc.VectorSubcoreMesh(core_axis_name="c", subcore_axis_name="s"),
    scratch_types=[
        pltpu.VMEM((CH, ROWS), jnp.int32),
        pltpu.VMEM((NBUF, ROWS, D), jnp.float32),
        pltpu.VMEM((HALF * D,), jnp.float32),
        pltpu.SemaphoreType.DMA,
        pltpu.SemaphoreType.DMA,
        pltpu.SemaphoreType.DMA,
        pltpu.SemaphoreType.DMA,
    ],
)(_sc_body)


BLKO = 2048


def _onehot_body(ids_ref, tab_ref, out_ref):
    g = pl.program_id(1)
    ids = ids_ref[0]
    iota = lax.broadcasted_iota(jnp.int32, (BLKO, VP), 1)
    oh = jnp.where(iota == ids, 1.0, 0.0).astype(jnp.bfloat16)
    contrib = jnp.dot(oh, tab_ref[0], preferred_element_type=jnp.float32)

    @pl.when(g == 0)
    def _():
        out_ref[...] = contrib

    @pl.when(g > 0)
    def _():
        out_ref[...] = out_ref[...] + contrib


def _onehot_embed(ids_tc, tab_tc):
    return pl.pallas_call(
        _onehot_body,
        grid=(B // BLKO, NTC),
        in_specs=[
            pl.BlockSpec((1, BLKO, 1), lambda i, g: (g, i, 0)),
            pl.BlockSpec((1, VP, D), lambda i, g: (g, 0, 0)),
        ],
        out_specs=pl.BlockSpec((BLKO, D), lambda i, g: (i, 0)),
        out_shape=jax.ShapeDtypeStruct((B, D), jnp.float32),
        compiler_params=pltpu.CompilerParams(
            dimension_semantics=("parallel", "arbitrary")
        ),
    )(ids_tc, tab_tc)


BLK = 2048


def _dense_body(cont_ref, csc_ref, ctc_ref, wc_ref, bc_ref, wo_ref, bo_ref, out_ref):
    c1 = jnp.dot(cont_ref[...], wc_ref[...], preferred_element_type=jnp.float32)
    c1 = c1 + bc_ref[...]
    wo = wo_ref[...]
    cat = csc_ref[...] + ctc_ref[...]
    h = jnp.dot(c1, wo[:D], preferred_element_type=jnp.float32)
    h = h + jnp.dot(cat, wo[D:], preferred_element_type=jnp.float32)
    out_ref[...] = h + bo_ref[...]


def _dense(cont, cat_sc, cat_tc, W_cont, b_cont, W_out, b_out):
    return pl.pallas_call(
        _dense_body,
        grid=(B // BLK,),
        in_specs=[
            pl.BlockSpec((BLK, N_CONT), lambda i: (i, 0)),
            pl.BlockSpec((BLK, D), lambda i: (i, 0)),
            pl.BlockSpec((BLK, D), lambda i: (i, 0)),
            pl.BlockSpec((N_CONT, D), lambda i: (0, 0)),
            pl.BlockSpec((1, D), lambda i: (0, 0)),
            pl.BlockSpec((2 * D, D), lambda i: (0, 0)),
            pl.BlockSpec((1, D), lambda i: (0, 0)),
        ],
        out_specs=pl.BlockSpec((BLK, D), lambda i: (i, 0)),
        out_shape=jax.ShapeDtypeStruct((B, D), jnp.float32),
    )(cont, cat_sc, cat_tc, W_cont, b_cont, W_out, b_out)


def kernel(x, W_cont, b_cont, tables, W_out, b_out):
    x = x.astype(jnp.int32)
    cont = x[:, :N_CONT].astype(jnp.float32)
    offs = 1 + V1 * jnp.arange(NSC, dtype=jnp.int32)
    fidx = (x[:, N_CONT : N_CONT + NSC] + offs).reshape(NW, CH, ROWS)
    tab_flat = tables[:NSC].reshape(NSC * V1, D)
    ids_tc = (1 + x[:, N_CONT + NSC :]).T.reshape(NTC, B, 1)
    tab_tc = jnp.pad(tables[NSC:], ((0, 0), (0, VP - V1), (0, 0))).astype(
        jnp.bfloat16
    )
    cat_sc = _sc_embed(tab_flat, fidx).reshape(B, D)
    cat_tc = _onehot_embed(ids_tc, tab_tc)
    return _dense(
        cont,
        cat_sc,
        cat_tc,
        W_cont,
        b_cont.reshape(1, D),
        W_out,
        b_out.reshape(1, D),
    )


# speedup vs baseline: 1.3919x; 1.2394x over previous
import functools

import jax
import jax.numpy as jnp
from jax import lax
from jax.experimental import pallas as pl
from jax.experimental.pallas import tpu as pltpu
from jax.experimental.pallas import tpu_sc as plsc

B = 16384
N_CONT = 13
N_CAT = 26
V1 = 1001
D = 128

NSC = 18
NTC = N_CAT - NSC
VP = 1024

NC = 2
NS = 16
NW = NC * NS
BPW = B // NW
S = 4
ROWS = S * NSC
CH = BPW // S
NBUF = 2


def _sc_body(tab_hbm, fidx_hbm, out_hbm, idx_v, ring_v, out_v, sem0, sem1):
    wid = lax.axis_index("s") * NC + lax.axis_index("c")
    pltpu.sync_copy(fidx_hbm.at[wid], idx_v)
    sems = (sem0, sem1)

    def start(c, buf):
        pltpu.async_copy(tab_hbm.at[idx_v.at[c]], ring_v.at[buf], sems[buf])

    def wait(buf):
        pltpu.make_async_copy(
            tab_hbm.at[idx_v.at[0]], ring_v.at[buf], sems[buf]
        ).wait()

    def accum(c, buf):
        for s in range(S):
            acc = [ring_v[buf, s * NSC, pl.ds(v * 16, 16)] for v in range(8)]
            for i in range(1, NSC):
                for v in range(8):
                    acc[v] = acc[v] + ring_v[buf, s * NSC + i, pl.ds(v * 16, 16)]
            row = pl.multiple_of((c * S + s) * D, 128)
            for v in range(8):
                out_v[pl.ds(row + v * 16, 16)] = acc[v]

    for b in range(NBUF):
        start(b, b)

    def rot(p, carry):
        c0 = NBUF * p
        for b in range(NBUF):
            wait(b)
            accum(c0 + b, b)

            @pl.when(c0 + b + NBUF < CH)
            def _():
                start(c0 + b + NBUF, b)

        return carry

    lax.fori_loop(0, CH // NBUF, rot, None)
    pltpu.sync_copy(out_v, out_hbm.at[wid])


_sc_embed = functools.partial(
    pl.kernel,
    out_type=jax.ShapeDtypeStruct((NW, BPW * D), jnp.float32),
    mesh=plsc.VectorSubcoreMesh(core_axis_name="c", subcore_axis_name="s"),
    scratch_types=[
        pltpu.VMEM((CH, ROWS), jnp.int32),
        pltpu.VMEM((NBUF, ROWS, D), jnp.float32),
        pltpu.VMEM((BPW * D,), jnp.float32),
        pltpu.SemaphoreType.DMA,
        pltpu.SemaphoreType.DMA,
    ],
)(_sc_body)


BLKO = 4096


def _onehot_body(ids_ref, tab_ref, out_ref):
    g = pl.program_id(1)
    ids = ids_ref[0]
    iota = lax.broadcasted_iota(jnp.int32, (BLKO, VP), 1)
    oh = jnp.where(iota == ids, 1.0, 0.0).astype(jnp.bfloat16)
    contrib = jnp.dot(oh, tab_ref[0], preferred_element_type=jnp.float32)

    @pl.when(g == 0)
    def _():
        out_ref[...] = contrib

    @pl.when(g > 0)
    def _():
        out_ref[...] = out_ref[...] + contrib


def _onehot_embed(ids_tc, tab_tc):
    return pl.pallas_call(
        _onehot_body,
        grid=(B // BLKO, NTC),
        in_specs=[
            pl.BlockSpec((1, BLKO, 1), lambda i, g: (g, i, 0)),
            pl.BlockSpec((1, VP, D), lambda i, g: (g, 0, 0)),
        ],
        out_specs=pl.BlockSpec((BLKO, D), lambda i, g: (i, 0)),
        out_shape=jax.ShapeDtypeStruct((B, D), jnp.float32),
        compiler_params=pltpu.CompilerParams(
            dimension_semantics=("parallel", "arbitrary")
        ),
    )(ids_tc, tab_tc)


BLK = 2048


def _dense_body(cont_ref, csc_ref, ctc_ref, wc_ref, bc_ref, wo_ref, bo_ref, out_ref):
    c1 = jnp.dot(cont_ref[...], wc_ref[...], preferred_element_type=jnp.float32)
    c1 = c1 + bc_ref[...]
    wo = wo_ref[...]
    cat = csc_ref[...] + ctc_ref[...]
    h = jnp.dot(c1, wo[:D], preferred_element_type=jnp.float32)
    h = h + jnp.dot(cat, wo[D:], preferred_element_type=jnp.float32)
    out_ref[...] = h + bo_ref[...]


def _dense(cont, cat_sc, cat_tc, W_cont, b_cont, W_out, b_out):
    return pl.pallas_call(
        _dense_body,
        grid=(B // BLK,),
        in_specs=[
            pl.BlockSpec((BLK, N_CONT), lambda i: (i, 0)),
            pl.BlockSpec((BLK, D), lambda i: (i, 0)),
            pl.BlockSpec((BLK, D), lambda i: (i, 0)),
            pl.BlockSpec((N_CONT, D), lambda i: (0, 0)),
            pl.BlockSpec((1, D), lambda i: (0, 0)),
            pl.BlockSpec((2 * D, D), lambda i: (0, 0)),
            pl.BlockSpec((1, D), lambda i: (0, 0)),
        ],
        out_specs=pl.BlockSpec((BLK, D), lambda i: (i, 0)),
        out_shape=jax.ShapeDtypeStruct((B, D), jnp.float32),
    )(cont, cat_sc, cat_tc, W_cont, b_cont, W_out, b_out)


def kernel(x, W_cont, b_cont, tables, W_out, b_out):
    x = x.astype(jnp.int32)
    cont = x[:, :N_CONT].astype(jnp.float32)
    offs = 1 + V1 * jnp.arange(NSC, dtype=jnp.int32)
    fidx = (x[:, N_CONT : N_CONT + NSC] + offs).reshape(NW, CH, ROWS)
    tab_flat = tables[:NSC].reshape(NSC * V1, D)
    ids_tc = (1 + x[:, N_CONT + NSC :]).T.reshape(NTC, B, 1)
    tab_tc = jnp.pad(tables[NSC:], ((0, 0), (0, VP - V1), (0, 0))).astype(
        jnp.bfloat16
    )
    cat_sc = _sc_embed(tab_flat, fidx).reshape(B, D)
    cat_tc = _onehot_embed(ids_tc, tab_tc)
    return _dense(
        cont,
        cat_sc,
        cat_tc,
        W_cont,
        b_cont.reshape(1, D),
        W_out,
        b_out.reshape(1, D),
    )
